# baseline (speedup 1.0000x reference)
import jax
import jax.numpy as jnp
from jax.experimental import pallas as pl

_MAX_LEN = 8192
_HIDDEN = 128
_BLK = 8192


def _body(in_ref, out_ref):
    out_ref[...] = in_ref[...]


def kernel(x, emb_table):
    seq_len = x.shape[1]
    out = pl.pallas_call(
        _body,
        grid=(_MAX_LEN // _BLK,),
        in_specs=[pl.BlockSpec((_BLK, _HIDDEN), lambda i: (i, 0))],
        out_specs=pl.BlockSpec((_BLK, _HIDDEN), lambda i: (i, 0)),
        out_shape=jax.ShapeDtypeStruct((_MAX_LEN, _HIDDEN), jnp.float32),
    )(emb_table)
    return out[None, :seq_len]
